# baseline (speedup 1.0000x reference)
import functools

import jax
import jax.numpy as jnp
from jax import lax
from jax.experimental import pallas as pl
from jax.experimental.pallas import tpu as pltpu
from jax.experimental.pallas import tpu_sc as plsc

GRID = 8
NB = 32.0
NCELLS = GRID * GRID
INV_CELL = 1.0 / (2.0 * NB / GRID)

NC, NS, L = 2, 16, 16
NW = NC * NS

def _lane_bcast(x, idx):
    dnums = lax.GatherDimensionNumbers(
        offset_dims=(), collapsed_slice_dims=(0,), start_index_map=(0,))
    return lax.gather(x, idx[:, None], dnums, slice_sizes=(1,),
                      mode=lax.GatherScatterMode.PROMISE_IN_BOUNDS)


NI = 8
CH = 128
GR = (NCELLS + 1) * 128


def _sc_bin_body(sx_hbm, sy_hbm, hid_hbm, out_hbm, sxv, syv, hbuf,
                 grids, *, a, h, aw):
    wid = lax.axis_index("s") * NC + lax.axis_index("c")
    pltpu.sync_copy(sx_hbm, sxv)
    pltpu.sync_copy(sy_hbm, syv)
    hiota = lax.iota(jnp.int32, L)
    nch = a // CH
    npg = CH // L

    def ib_body(ib, _):
        def zero_body(k, _):
            grids[pl.ds(k * L, L)] = jnp.zeros((L,), jnp.float32)
            return 0
        lax.fori_loop(0, NI * GR // L, zero_body, 0)

        i_base = wid * aw + ib * NI

        def ch_body(ch, _):
            pltpu.sync_copy(hid_hbm.at[pl.ds(ch * CH, CH), :], hbuf)

            def il_body(il, _):
                i = i_base + il
                lane = hiota * 0 + (i % L)
                xg = sxv[pl.ds((i // L) * L, L)]
                yg = syv[pl.ds((i // L) * L, L)]
                xi = _lane_bcast(xg, lane)
                yi = _lane_bcast(yg, lane)
                gbase = hiota + il * GR

                def jg_body(jg, _):
                    j0 = ch * CH + jg * L
                    xj = sxv[pl.ds(j0, L)]
                    yj = syv[pl.ds(j0, L)]
                    rx = xj - xi
                    ry = yj - yi
                    col = ((rx + NB) * INV_CELL).astype(jnp.int32)
                    row = ((ry + NB) * INV_CELL).astype(jnp.int32)
                    ok = (jnp.abs(rx) < NB) & (jnp.abs(ry) < NB)
                    ok = ok & ((hiota + j0) != i)
                    cell = (row << 3) + col
                    cell = jnp.where(ok, cell, NCELLS)
                    cell128 = cell << 7

                    @plsc.parallel_loop(0, L, 1, unroll=8)
                    def p_body(p):
                        cb = _lane_bcast(cell128, hiota * 0 + p)
                        idx0 = cb + gbase
                        jrow = jg * L + p
                        for s in range(h // L):
                            val = hbuf[jrow, pl.ds(s * L, L)]
                            plsc.addupdate_scatter(grids, [idx0 + s * L], val)
                    return 0
                lax.fori_loop(0, npg, jg_body, 0)
                return 0
            lax.fori_loop(0, NI, il_body, 0)
            return 0
        lax.fori_loop(0, nch, ch_body, 0)

        def drain_body(il, _):
            pltpu.sync_copy(grids.at[pl.ds(il * GR, NCELLS * h)],
                            out_hbm.at[i_base + il])
            return 0
        lax.fori_loop(0, NI, drain_body, 0)
        return 0
    lax.fori_loop(0, aw // NI, ib_body, 0)


def _sc_bin(sx, sy, hidden):
    a, h = hidden.shape
    aw = a // NW
    mesh = plsc.VectorSubcoreMesh(core_axis_name="c", subcore_axis_name="s")
    return pl.kernel(
        functools.partial(_sc_bin_body, a=a, h=h, aw=aw),
        out_type=jax.ShapeDtypeStruct((a, NCELLS * h), jnp.float32),
        mesh=mesh,
        compiler_params=pltpu.CompilerParams(needs_layout_passes=False),
        scratch_types=[
            pltpu.VMEM((a,), jnp.float32),
            pltpu.VMEM((a,), jnp.float32),
            pltpu.VMEM((CH, h), jnp.float32),
            pltpu.VMEM((NI * GR,), jnp.float32),
        ],
    )(sx, sy, hidden)


def _mm_body(g_ref, wt_ref, b2_ref, mc_ref, out_ref):
    gb = g_ref[...].astype(jnp.bfloat16)
    acc = jnp.dot(gb, wt_ref[...], preferred_element_type=jnp.float32)
    out_ref[...] = (acc + b2_ref[...]) * mc_ref[...]


def _tc_matmul(grids, wt, b2, maskc):
    a = grids.shape[0]
    h = wt.shape[1]
    bi = 256 if a % 256 == 0 else a
    return pl.pallas_call(
        _mm_body,
        grid=(a // bi,),
        in_specs=[
            pl.BlockSpec((bi, NCELLS * h), lambda i: (i, 0)),
            pl.BlockSpec((NCELLS * h, h), lambda i: (0, 0)),
            pl.BlockSpec((1, h), lambda i: (0, 0)),
            pl.BlockSpec((bi, 1), lambda i: (i, 0)),
        ],
        out_specs=pl.BlockSpec((bi, h), lambda i: (i, 0)),
        out_shape=jax.ShapeDtypeStruct((a, h), jnp.float32),
    )(grids, wt, b2, maskc)


def kernel(hidden, pos, mask, W, b):
    a, h = hidden.shape
    mask_f = mask.astype(jnp.float32)
    big = jnp.float32(1e30)
    sx = jnp.where(mask, pos[:, 0], big)
    sy = jnp.where(mask, pos[:, 1], big)
    grids = _sc_bin(sx, sy, hidden)
    wt = W.T.astype(jnp.bfloat16)
    b2 = b.reshape(1, h)
    maskc = mask_f.reshape(a, 1)
    return _tc_matmul(grids, wt, b2, maskc)


# speedup vs baseline: 7.7397x; 7.7397x over previous
import functools

import jax
import jax.numpy as jnp
from jax import lax
from jax.experimental import pallas as pl
from jax.experimental.pallas import tpu as pltpu
from jax.experimental.pallas import tpu_sc as plsc

GRID = 8
NB = 32.0
NCELLS = GRID * GRID
INV_CELL = 1.0 / (2.0 * NB / GRID)

NC, NS, L = 2, 16, 16
NW = NC * NS

def _lane_bcast(x, idx):
    dnums = lax.GatherDimensionNumbers(
        offset_dims=(), collapsed_slice_dims=(0,), start_index_map=(0,))
    return lax.gather(x, idx[:, None], dnums, slice_sizes=(1,),
                      mode=lax.GatherScatterMode.PROMISE_IN_BOUNDS)


NI = 8
CH = 128
GR = (NCELLS + 1) * 128


def _sc_bin_body(sx_hbm, sy_hbm, hid_hbm, out_hbm, sxv, syv, hbuf,
                 grids, *, a, h, aw):
    wid = lax.axis_index("s") * NC + lax.axis_index("c")
    pltpu.sync_copy(sx_hbm, sxv)
    pltpu.sync_copy(sy_hbm, syv)
    hiota = lax.iota(jnp.int32, L)
    nch = a // CH
    npg = CH // L

    def ib_body(ib, _):
        def zero_body(k, _):
            grids[pl.ds(k * L, L)] = jnp.zeros((L,), jnp.float32)
            return 0
        lax.fori_loop(0, NI * GR // L, zero_body, 0)

        i_base = wid * aw + ib * NI

        def ch_body(ch, _):
            pltpu.sync_copy(hid_hbm.at[pl.ds(ch * CH, CH), :], hbuf)

            def il_body(il, _):
                i = i_base + il
                lane = hiota * 0 + (i % L)
                xg = sxv[pl.ds((i // L) * L, L)]
                yg = syv[pl.ds((i // L) * L, L)]
                xi = _lane_bcast(xg, lane)
                yi = _lane_bcast(yg, lane)
                gbase = hiota + il * GR

                def jg_body(jg, _):
                    j0 = ch * CH + jg * L
                    xj = sxv[pl.ds(j0, L)]
                    yj = syv[pl.ds(j0, L)]
                    rx = xj - xi
                    ry = yj - yi
                    col = ((rx + NB) * INV_CELL).astype(jnp.int32)
                    row = ((ry + NB) * INV_CELL).astype(jnp.int32)
                    ok = (jnp.abs(rx) < NB) & (jnp.abs(ry) < NB)
                    ok = ok & ((hiota + j0) != i)
                    cell = (row << 3) + col
                    cell = jnp.where(ok, cell, NCELLS)
                    cell128 = cell << 7

                    @plsc.parallel_loop(0, L, 1, unroll=4)
                    def p_body(p):
                        cb = _lane_bcast(cell128, hiota * 0 + p)
                        idx0 = cb + gbase
                        jrow = jg * L + p
                        for s in range(h // L):
                            val = hbuf[jrow, pl.ds(s * L, L)]
                            plsc.addupdate_scatter(grids, [idx0 + s * L], val)
                    return 0
                lax.fori_loop(0, npg, jg_body, 0)
                return 0
            lax.fori_loop(0, NI, il_body, 0)
            return 0
        lax.fori_loop(0, nch, ch_body, 0)

        def drain_body(il, _):
            pltpu.sync_copy(grids.at[pl.ds(il * GR, NCELLS * h)],
                            out_hbm.at[i_base + il])
            return 0
        lax.fori_loop(0, NI, drain_body, 0)
        return 0
    lax.fori_loop(0, aw // NI, ib_body, 0)


def _sc_bin(sx, sy, hidden, n_sc):
    a, h = hidden.shape
    aw = n_sc // NW
    mesh = plsc.VectorSubcoreMesh(core_axis_name="c", subcore_axis_name="s")
    return pl.kernel(
        functools.partial(_sc_bin_body, a=a, h=h, aw=aw),
        out_type=jax.ShapeDtypeStruct((n_sc, NCELLS * h), jnp.float32),
        mesh=mesh,
        compiler_params=pltpu.CompilerParams(needs_layout_passes=False),
        scratch_types=[
            pltpu.VMEM((a,), jnp.float32),
            pltpu.VMEM((a,), jnp.float32),
            pltpu.VMEM((CH, h), jnp.float32),
            pltpu.VMEM((NI * GR,), jnp.float32),
        ],
    )(sx, sy, hidden)


def _tc_bin_body(pxr, pyr, maskr, pxc, pyc, maskc, hid, wt, b2, out_ref,
                 scratch, *, i_off, bi, a, h):
    i0 = i_off + pl.program_id(0) * bi
    px_i = pxc[pl.ds(i0, bi), :]
    py_i = pyc[pl.ds(i0, bi), :]
    m_i = maskc[pl.ds(i0, bi), :]

    relx = pxr[...] - px_i
    rely = pyr[...] - py_i

    colf = jnp.clip(jnp.floor((relx + NB) * INV_CELL), 0.0, GRID - 1.0)
    rowf = jnp.clip(jnp.floor((rely + NB) * INV_CELL), 0.0, GRID - 1.0)
    within = (jnp.abs(relx) < NB) & (jnp.abs(rely) < NB)
    jr = jax.lax.broadcasted_iota(jnp.int32, (bi, a), 1)
    ir = jax.lax.broadcasted_iota(jnp.int32, (bi, a), 0) + i0
    valid = within & (jr != ir)
    vm = jnp.where(valid, 1.0, 0.0) * maskr[...]

    row_oh = [(jnp.where(rowf == float(r), 1.0, 0.0) * vm).astype(jnp.bfloat16)
              for r in range(GRID)]
    col_oh = [jnp.where(colf == float(c), 1.0, 0.0).astype(jnp.bfloat16)
              for c in range(GRID)]

    hid_b = hid[...]
    for cell in range(NCELLS):
        r, c = cell // GRID, cell % GRID
        oh = row_oh[r] * col_oh[c]
        grid_c = jnp.dot(oh, hid_b, preferred_element_type=jnp.float32)
        scratch[:, cell * h:(cell + 1) * h] = grid_c.astype(jnp.bfloat16)

    acc = jnp.dot(scratch[...], wt[...], preferred_element_type=jnp.float32)
    out_ref[...] = (acc + b2[...]) * m_i


def _tc_bin(pos, mask_f, hid_b, wt, b2, i_off, n_rows):
    a, h = hid_b.shape
    bi = 256
    pxr = pos[:, 0].reshape(1, a)
    pyr = pos[:, 1].reshape(1, a)
    maskr = mask_f.reshape(1, a)
    pxc = pos[:, 0].reshape(a, 1)
    pyc = pos[:, 1].reshape(a, 1)
    maskc = mask_f.reshape(a, 1)

    full = lambda s: pl.BlockSpec(s, lambda i: tuple(0 for _ in s))
    return pl.pallas_call(
        functools.partial(_tc_bin_body, i_off=i_off, bi=bi, a=a, h=h),
        grid=(n_rows // bi,),
        in_specs=[
            full((1, a)), full((1, a)), full((1, a)),
            full((a, 1)), full((a, 1)), full((a, 1)),
            full((a, h)),
            full((NCELLS * h, h)),
            full((1, h)),
        ],
        out_specs=pl.BlockSpec((bi, h), lambda i: (i, 0)),
        out_shape=jax.ShapeDtypeStruct((n_rows, h), jnp.float32),
        scratch_shapes=[pltpu.VMEM((bi, NCELLS * h), jnp.bfloat16)],
    )(pxr, pyr, maskr, pxc, pyc, maskc, hid_b, wt, b2)


def _mm_body(g_ref, wt_ref, b2_ref, mc_ref, out_ref):
    gb = g_ref[...].astype(jnp.bfloat16)
    acc = jnp.dot(gb, wt_ref[...], preferred_element_type=jnp.float32)
    out_ref[...] = (acc + b2_ref[...]) * mc_ref[...]


def _tc_matmul(grids, wt, b2, maskc):
    a = grids.shape[0]
    h = wt.shape[1]
    bi = 256 if a % 256 == 0 else a
    return pl.pallas_call(
        _mm_body,
        grid=(a // bi,),
        in_specs=[
            pl.BlockSpec((bi, NCELLS * h), lambda i: (i, 0)),
            pl.BlockSpec((NCELLS * h, h), lambda i: (0, 0)),
            pl.BlockSpec((1, h), lambda i: (0, 0)),
            pl.BlockSpec((bi, 1), lambda i: (i, 0)),
        ],
        out_specs=pl.BlockSpec((bi, h), lambda i: (i, 0)),
        out_shape=jax.ShapeDtypeStruct((a, h), jnp.float32),
    )(grids, wt, b2, maskc)


def kernel(hidden, pos, mask, W, b):
    a, h = hidden.shape
    mask_f = mask.astype(jnp.float32)
    wt = W.T.astype(jnp.bfloat16)
    b2 = b.reshape(1, h)
    hid_b = hidden.astype(jnp.bfloat16)
    s = 256 if (a % 256 == 0 and a >= 512) else 0
    big = jnp.float32(1e30)
    sx = jnp.where(mask, pos[:, 0], big)
    sy = jnp.where(mask, pos[:, 1], big)
    out_tc = _tc_bin(pos, mask_f, hid_b, wt, b2, s, a - s)
    if s == 0:
        return out_tc
    grids = _sc_bin(sx, sy, hidden, s)
    maskc = mask_f[:s].reshape(s, 1)
    out_sc = _tc_matmul(grids, wt, b2, maskc)
    return jnp.concatenate([out_sc, out_tc], axis=0)


# speedup vs baseline: 9.8056x; 1.2669x over previous
import functools

import jax
import jax.numpy as jnp
from jax import lax
from jax.experimental import pallas as pl
from jax.experimental.pallas import tpu as pltpu
from jax.experimental.pallas import tpu_sc as plsc

GRID = 8
NB = 32.0
NCELLS = GRID * GRID
INV_CELL = 1.0 / (2.0 * NB / GRID)

NC, NS, L = 2, 16, 16
NW = NC * NS

def _lane_bcast(x, idx):
    dnums = lax.GatherDimensionNumbers(
        offset_dims=(), collapsed_slice_dims=(0,), start_index_map=(0,))
    return lax.gather(x, idx[:, None], dnums, slice_sizes=(1,),
                      mode=lax.GatherScatterMode.PROMISE_IN_BOUNDS)


NI = 8
CH = 128
GR = (NCELLS + 1) * 128


def _sc_bin_body(sx_hbm, sy_hbm, hid_hbm, out_hbm, sxv, syv, hbuf,
                 grids, *, a, h, aw, ni):
    wid = lax.axis_index("s") * NC + lax.axis_index("c")
    pltpu.sync_copy(sx_hbm, sxv)
    pltpu.sync_copy(sy_hbm, syv)
    hiota = lax.iota(jnp.int32, L)
    nch = a // CH
    npg = CH // L

    def ib_body(ib, _):
        def zero_body(k, _):
            grids[pl.ds(k * L, L)] = jnp.zeros((L,), jnp.float32)
            return 0
        lax.fori_loop(0, ni * GR // L, zero_body, 0)

        i_base = wid * aw + ib * ni

        def ch_body(ch, _):
            pltpu.sync_copy(hid_hbm.at[pl.ds(ch * CH, CH), :], hbuf)

            def il_body(il, _):
                i = i_base + il
                lane = hiota * 0 + (i % L)
                xg = sxv[pl.ds((i // L) * L, L)]
                yg = syv[pl.ds((i // L) * L, L)]
                xi = _lane_bcast(xg, lane)
                yi = _lane_bcast(yg, lane)
                gbase = hiota + il * GR

                def jg_body(jg, _):
                    j0 = ch * CH + jg * L
                    xj = sxv[pl.ds(j0, L)]
                    yj = syv[pl.ds(j0, L)]
                    rx = xj - xi
                    ry = yj - yi
                    col = ((rx + NB) * INV_CELL).astype(jnp.int32)
                    row = ((ry + NB) * INV_CELL).astype(jnp.int32)
                    ok = (jnp.abs(rx) < NB) & (jnp.abs(ry) < NB)
                    ok = ok & ((hiota + j0) != i)
                    cell = (row << 3) + col
                    cell = jnp.where(ok, cell, NCELLS)
                    cell128 = cell << 7

                    @plsc.parallel_loop(0, L, 1, unroll=4)
                    def p_body(p):
                        cb = _lane_bcast(cell128, hiota * 0 + p)
                        idx0 = cb + gbase
                        jrow = jg * L + p
                        for s in range(h // L):
                            val = hbuf[jrow, pl.ds(s * L, L)]
                            plsc.addupdate_scatter(grids, [idx0 + s * L], val)
                    return 0
                lax.fori_loop(0, npg, jg_body, 0)
                return 0
            lax.fori_loop(0, ni, il_body, 0)
            return 0
        lax.fori_loop(0, nch, ch_body, 0)

        def drain_body(il, _):
            pltpu.sync_copy(grids.at[pl.ds(il * GR, NCELLS * h)],
                            out_hbm.at[i_base + il])
            return 0
        lax.fori_loop(0, ni, drain_body, 0)
        return 0
    lax.fori_loop(0, aw // ni, ib_body, 0)


def _sc_bin(sx, sy, hidden, n_sc):
    a, h = hidden.shape
    aw = n_sc // NW
    mesh = plsc.VectorSubcoreMesh(core_axis_name="c", subcore_axis_name="s")
    return pl.kernel(
        functools.partial(_sc_bin_body, a=a, h=h, aw=aw, ni=min(NI, aw)),
        out_type=jax.ShapeDtypeStruct((n_sc, NCELLS * h), jnp.float32),
        mesh=mesh,
        compiler_params=pltpu.CompilerParams(needs_layout_passes=False),
        scratch_types=[
            pltpu.VMEM((a,), jnp.float32),
            pltpu.VMEM((a,), jnp.float32),
            pltpu.VMEM((CH, h), jnp.float32),
            pltpu.VMEM((NI * GR,), jnp.float32),
        ],
    )(sx, sy, hidden)


def _tc_bin_body(pxr, pyr, maskr, pxc, pyc, maskc, hid, wt, b2, out_ref,
                 scratch, *, i_off, bi, a, h):
    i0 = i_off + pl.program_id(0) * bi
    px_i = pxc[pl.ds(i0, bi), :]
    py_i = pyc[pl.ds(i0, bi), :]
    m_i = maskc[pl.ds(i0, bi), :]

    relx = pxr[...] - px_i
    rely = pyr[...] - py_i

    colf = jnp.clip(jnp.floor((relx + NB) * INV_CELL), 0.0, GRID - 1.0)
    rowf = jnp.clip(jnp.floor((rely + NB) * INV_CELL), 0.0, GRID - 1.0)
    within = (jnp.abs(relx) < NB) & (jnp.abs(rely) < NB)
    jr = jax.lax.broadcasted_iota(jnp.int32, (bi, a), 1)
    ir = jax.lax.broadcasted_iota(jnp.int32, (bi, a), 0) + i0
    valid = within & (jr != ir)
    vm = jnp.where(valid, 1.0, 0.0) * maskr[...]

    row_oh = [(jnp.where(rowf == float(r), 1.0, 0.0) * vm).astype(jnp.bfloat16)
              for r in range(GRID)]
    col_oh = [jnp.where(colf == float(c), 1.0, 0.0).astype(jnp.bfloat16)
              for c in range(GRID)]

    hid_b = hid[...]
    for cell in range(NCELLS):
        r, c = cell // GRID, cell % GRID
        oh = row_oh[r] * col_oh[c]
        grid_c = jnp.dot(oh, hid_b, preferred_element_type=jnp.float32)
        scratch[:, cell * h:(cell + 1) * h] = grid_c.astype(jnp.bfloat16)

    acc = jnp.dot(scratch[...], wt[...], preferred_element_type=jnp.float32)
    out_ref[...] = (acc + b2[...]) * m_i


def _tc_bin(pos, mask_f, hid_b, wt, b2, i_off, n_rows):
    a, h = hid_b.shape
    bi = 256 if n_rows % 256 == 0 else 128
    pxr = pos[:, 0].reshape(1, a)
    pyr = pos[:, 1].reshape(1, a)
    maskr = mask_f.reshape(1, a)
    pxc = pos[:, 0].reshape(a, 1)
    pyc = pos[:, 1].reshape(a, 1)
    maskc = mask_f.reshape(a, 1)

    full = lambda s: pl.BlockSpec(s, lambda i: tuple(0 for _ in s))
    return pl.pallas_call(
        functools.partial(_tc_bin_body, i_off=i_off, bi=bi, a=a, h=h),
        grid=(n_rows // bi,),
        in_specs=[
            full((1, a)), full((1, a)), full((1, a)),
            full((a, 1)), full((a, 1)), full((a, 1)),
            full((a, h)),
            full((NCELLS * h, h)),
            full((1, h)),
        ],
        out_specs=pl.BlockSpec((bi, h), lambda i: (i, 0)),
        out_shape=jax.ShapeDtypeStruct((n_rows, h), jnp.float32),
        scratch_shapes=[pltpu.VMEM((bi, NCELLS * h), jnp.bfloat16)],
    )(pxr, pyr, maskr, pxc, pyc, maskc, hid_b, wt, b2)


def _mm_body(g_ref, wt_ref, b2_ref, mc_ref, out_ref):
    gb = g_ref[...].astype(jnp.bfloat16)
    acc = jnp.dot(gb, wt_ref[...], preferred_element_type=jnp.float32)
    out_ref[...] = (acc + b2_ref[...]) * mc_ref[...]


def _tc_matmul(grids, wt, b2, maskc):
    a = grids.shape[0]
    h = wt.shape[1]
    bi = 256 if a % 256 == 0 else a
    return pl.pallas_call(
        _mm_body,
        grid=(a // bi,),
        in_specs=[
            pl.BlockSpec((bi, NCELLS * h), lambda i: (i, 0)),
            pl.BlockSpec((NCELLS * h, h), lambda i: (0, 0)),
            pl.BlockSpec((1, h), lambda i: (0, 0)),
            pl.BlockSpec((bi, 1), lambda i: (i, 0)),
        ],
        out_specs=pl.BlockSpec((bi, h), lambda i: (i, 0)),
        out_shape=jax.ShapeDtypeStruct((a, h), jnp.float32),
    )(grids, wt, b2, maskc)


def kernel(hidden, pos, mask, W, b):
    a, h = hidden.shape
    mask_f = mask.astype(jnp.float32)
    wt = W.T.astype(jnp.bfloat16)
    b2 = b.reshape(1, h)
    hid_b = hidden.astype(jnp.bfloat16)
    s = 128 if (a % 256 == 0 and a >= 512) else 0
    big = jnp.float32(1e30)
    sx = jnp.where(mask, pos[:, 0], big)
    sy = jnp.where(mask, pos[:, 1], big)
    out_tc = _tc_bin(pos, mask_f, hid_b, wt, b2, s, a - s)
    if s == 0:
        return out_tc
    grids = _sc_bin(sx, sy, hidden, s)
    maskc = mask_f[:s].reshape(s, 1)
    out_sc = _tc_matmul(grids, wt, b2, maskc)
    return jnp.concatenate([out_sc, out_tc], axis=0)
